# Initial kernel scaffold; baseline (speedup 1.0000x reference)
import jax
import jax.numpy as jnp
from jax.experimental import pallas as pl


def kernel(h):
    raise NotImplementedError("write your pallas kernel here")


# speedup vs baseline: 9.4368x; 9.4368x over previous
import jax
import jax.numpy as jnp
from jax.experimental import pallas as pl
from jax.experimental.pallas import tpu as pltpu

K = 16
N_POINTS = 2048
N_DIMS = 256
ROW_BLOCK = 256


def _knn_block_kernel(x_row_ref, x_all_ref, idx_ref):
    x_r = x_row_ref[0]
    x_c = x_all_ref[0]
    dot = jax.lax.dot_general(
        x_r, x_c, (((1,), (1,)), ((), ())),
        preferred_element_type=jnp.float32)
    n_r = jnp.sum(x_r * x_r, axis=1, keepdims=True)
    n_c = jnp.sum(x_c * x_c, axis=1)
    d = (n_r + n_c[None, :]) - 2.0 * dot

    iota = jax.lax.broadcasted_iota(jnp.int32, (ROW_BLOCK, N_POINTS), 1)
    big_i = jnp.int32(2 ** 30)
    inf = jnp.float32(jnp.inf)
    x = d
    for k in range(K):
        m = jnp.min(x, axis=1, keepdims=True)
        eq = x == m
        idx = jnp.min(jnp.where(eq, iota, big_i), axis=1, keepdims=True)
        idx_ref[0, :, k] = idx[:, 0]
        x = jnp.where(iota == idx, inf, x)


def _knn_indices(h):
    n_samples = h.shape[0]
    grid = (n_samples, N_POINTS // ROW_BLOCK)
    return pl.pallas_call(
        _knn_block_kernel,
        grid=grid,
        in_specs=[
            pl.BlockSpec((1, ROW_BLOCK, N_DIMS), lambda s, r: (s, r, 0)),
            pl.BlockSpec((1, N_POINTS, N_DIMS), lambda s, r: (s, 0, 0)),
        ],
        out_specs=pl.BlockSpec((1, ROW_BLOCK, K), lambda s, r: (s, r, 0)),
        out_shape=jax.ShapeDtypeStruct((n_samples, N_POINTS, K), jnp.int32),
    )(h, h)


def kernel(h):
    n_samples, n_points, _ = h.shape
    k_indices = _knn_indices(h)
    src = jnp.repeat(jnp.arange(n_points, dtype=jnp.int32), K)
    dst = k_indices.reshape(n_samples, n_points * K)
    return (k_indices, src, dst)


# speedup vs baseline: 12.1769x; 1.2904x over previous
import jax
import jax.numpy as jnp
from jax.experimental import pallas as pl
from jax.experimental.pallas import tpu as pltpu

K = 16
N_POINTS = 2048
N_DIMS = 256
ROW_BLOCK = 256


def _knn_block_kernel(x_row_ref, x_all_ref, n_row_ref, n_all_ref, idx_ref):
    x_r = x_row_ref[0]
    x_c = x_all_ref[0]
    dot = jax.lax.dot_general(
        x_r, x_c, (((1,), (1,)), ((), ())),
        preferred_element_type=jnp.float32)
    n_r = n_row_ref[0]
    n_c = n_all_ref[0]
    d = (n_r + n_c) - 2.0 * dot

    iota_f = jax.lax.broadcasted_iota(
        jnp.int32, (ROW_BLOCK, N_POINTS), 1).astype(jnp.float32)
    big_f = jnp.float32(4096.0)
    inf = jnp.float32(jnp.inf)
    x = d
    for k in range(K):
        m = jnp.min(x, axis=1, keepdims=True)
        eq = x == m
        idx_f = jnp.min(jnp.where(eq, iota_f, big_f), axis=1, keepdims=True)
        idx_ref[0, :, k] = idx_f[:, 0].astype(jnp.int32)
        x = jnp.where(iota_f == idx_f, inf, x)


def _knn_indices(h):
    n_samples = h.shape[0]
    x2 = jnp.sum(h * h, axis=-1)
    x2_r = x2[:, :, None]
    x2_c = x2[:, None, :]
    grid = (n_samples, N_POINTS // ROW_BLOCK)
    return pl.pallas_call(
        _knn_block_kernel,
        grid=grid,
        in_specs=[
            pl.BlockSpec((1, ROW_BLOCK, N_DIMS), lambda s, r: (s, r, 0)),
            pl.BlockSpec((1, N_POINTS, N_DIMS), lambda s, r: (s, 0, 0)),
            pl.BlockSpec((1, ROW_BLOCK, 1), lambda s, r: (s, r, 0)),
            pl.BlockSpec((1, 1, N_POINTS), lambda s, r: (s, 0, 0)),
        ],
        out_specs=pl.BlockSpec((1, ROW_BLOCK, K), lambda s, r: (s, r, 0)),
        out_shape=jax.ShapeDtypeStruct((n_samples, N_POINTS, K), jnp.int32),
    )(h, h, x2_r, x2_c)


def kernel(h):
    n_samples, n_points, _ = h.shape
    k_indices = _knn_indices(h)
    src = jnp.repeat(jnp.arange(n_points, dtype=jnp.int32), K)
    dst = k_indices.reshape(n_samples, n_points * K)
    return (k_indices, src, dst)


# speedup vs baseline: 15.8746x; 1.3037x over previous
import jax
import jax.numpy as jnp
from jax.experimental import pallas as pl

K = 16
N_POINTS = 2048
N_DIMS = 256
ROW_BLOCK = 256
N_CHUNKS = 16
LANES = N_POINTS // N_CHUNKS
BIG = 4096.0


def _ce(a, b):
    av, ag = a
    bv, bg = b
    swap = bv < av
    lo = (jnp.where(swap, bv, av), jnp.where(swap, bg, ag))
    hi = (jnp.where(swap, av, bv), jnp.where(swap, ag, bg))
    return lo, hi


def _cmin(a, b):
    av, ag = a
    bv, bg = b
    swap = bv < av
    return (jnp.where(swap, bv, av), jnp.where(swap, bg, ag))


def _oemerge(a, b):
    if len(a) == 1:
        lo, hi = _ce(a[0], b[0])
        return [lo, hi]
    ev = _oemerge(a[0::2], b[0::2])
    od = _oemerge(a[1::2], b[1::2])
    out = [ev[0]]
    for i in range(len(od) - 1):
        lo, hi = _ce(od[i], ev[i + 1])
        out.extend((lo, hi))
    out.append(od[-1])
    return out


def _bitonic_sort8(x):
    x = list(x)
    for stride, idxs in ((4, (0, 1, 2, 3)), (2, (0, 1, 4, 5)),
                         (1, (0, 2, 4, 6))):
        for i in idxs:
            x[i], x[i + stride] = _ce(x[i], x[i + stride])
    return x


def _top8_per_lane(d, iota_f):
    lists = [[(d[:, c * LANES:(c + 1) * LANES],
               iota_f[:, c * LANES:(c + 1) * LANES])]
             for c in range(N_CHUNKS)]
    while len(lists) > 2:
        lists = [_oemerge(lists[i], lists[i + 1])
                 for i in range(0, len(lists), 2)]
    a, b = lists
    lows = [_cmin(a[i], b[7 - i]) for i in range(8)]
    return _bitonic_sort8(lows)


def _knn_block_kernel(x_row_ref, x_all_ref, n_row_ref, n_all_ref, idx_ref):
    x_r = x_row_ref[0]
    x_c = x_all_ref[0]
    dot = jax.lax.dot_general(
        x_r, x_c, (((1,), (1,)), ((), ())),
        preferred_element_type=jnp.float32)
    n_r = n_row_ref[0]
    n_c = n_all_ref[0]
    d = (n_r + n_c) - 2.0 * dot

    iota_f = jax.lax.broadcasted_iota(
        jnp.int32, (ROW_BLOCK, N_POINTS), 1).astype(jnp.float32)
    slots = _top8_per_lane(d, iota_f)
    sv = [s[0] for s in slots]
    sg = [s[1] for s in slots]
    inf = jnp.float32(jnp.inf)
    for k in range(K):
        m = jnp.min(sv[0], axis=1, keepdims=True)
        cand = jnp.where(sv[0] == m, sg[0], BIG)
        idx_f = jnp.min(cand, axis=1, keepdims=True)
        idx_ref[0, :, k] = idx_f[:, 0].astype(jnp.int32)
        w = cand == idx_f
        for s in range(7):
            sv[s] = jnp.where(w, sv[s + 1], sv[s])
            sg[s] = jnp.where(w, sg[s + 1], sg[s])
        sv[7] = jnp.where(w, inf, sv[7])
        sg[7] = jnp.where(w, BIG, sg[7])


def _knn_indices(h):
    n_samples = h.shape[0]
    x2 = jnp.sum(h * h, axis=-1)
    x2_r = x2[:, :, None]
    x2_c = x2[:, None, :]
    grid = (n_samples, N_POINTS // ROW_BLOCK)
    return pl.pallas_call(
        _knn_block_kernel,
        grid=grid,
        in_specs=[
            pl.BlockSpec((1, ROW_BLOCK, N_DIMS), lambda s, r: (s, r, 0)),
            pl.BlockSpec((1, N_POINTS, N_DIMS), lambda s, r: (s, 0, 0)),
            pl.BlockSpec((1, ROW_BLOCK, 1), lambda s, r: (s, r, 0)),
            pl.BlockSpec((1, 1, N_POINTS), lambda s, r: (s, 0, 0)),
        ],
        out_specs=pl.BlockSpec((1, ROW_BLOCK, K), lambda s, r: (s, r, 0)),
        out_shape=jax.ShapeDtypeStruct((n_samples, N_POINTS, K), jnp.int32),
    )(h, h, x2_r, x2_c)


def kernel(h):
    n_samples, n_points, _ = h.shape
    k_indices = _knn_indices(h)
    src = jnp.repeat(jnp.arange(n_points, dtype=jnp.int32), K)
    dst = k_indices.reshape(n_samples, n_points * K)
    return (k_indices, src, dst)


# speedup vs baseline: 16.7701x; 1.0564x over previous
import jax
import jax.numpy as jnp
from jax.experimental import pallas as pl

K = 16
N_POINTS = 2048
N_DIMS = 256
ROW_BLOCK = 256
N_CHUNKS = 16
LANES = N_POINTS // N_CHUNKS
BIG = 4096.0


def _ce(a, b):
    av, ag = a
    bv, bg = b
    swap = bv < av
    lo = (jnp.where(swap, bv, av), jnp.where(swap, bg, ag))
    hi = (jnp.where(swap, av, bv), jnp.where(swap, ag, bg))
    return lo, hi


def _cmin(a, b):
    av, ag = a
    bv, bg = b
    swap = bv < av
    return (jnp.where(swap, bv, av), jnp.where(swap, bg, ag))


def _oemerge(a, b):
    if len(a) == 1:
        lo, hi = _ce(a[0], b[0])
        return [lo, hi]
    ev = _oemerge(a[0::2], b[0::2])
    od = _oemerge(a[1::2], b[1::2])
    out = [ev[0]]
    for i in range(len(od) - 1):
        lo, hi = _ce(od[i], ev[i + 1])
        out.extend((lo, hi))
    out.append(od[-1])
    return out


def _bitonic_sort8(x):
    x = list(x)
    for stride, idxs in ((4, (0, 1, 2, 3)), (2, (0, 1, 4, 5)),
                         (1, (0, 2, 4, 6))):
        for i in idxs:
            x[i], x[i + stride] = _ce(x[i], x[i + stride])
    return x


def _top8_per_lane(d, iota_f):
    lists = [[(d[:, c * LANES:(c + 1) * LANES],
               iota_f[:, c * LANES:(c + 1) * LANES])]
             for c in range(N_CHUNKS)]
    while len(lists) > 2:
        lists = [_oemerge(lists[i], lists[i + 1])
                 for i in range(0, len(lists), 2)]
    a, b = lists
    lows = [_cmin(a[i], b[7 - i]) for i in range(8)]
    return _bitonic_sort8(lows)


def _knn_block_kernel(x_row_ref, x_all_ref, n_row_ref, n_all_ref, idx_ref):
    x_r = x_row_ref[0]
    x_c = x_all_ref[0]
    dot = jax.lax.dot_general(
        x_r, x_c, (((1,), (1,)), ((), ())),
        preferred_element_type=jnp.float32)
    n_r = n_row_ref[0]
    n_c = n_all_ref[0]
    d = (n_r + n_c) - 2.0 * dot

    iota_f = jax.lax.broadcasted_iota(
        jnp.int32, (ROW_BLOCK, N_POINTS), 1).astype(jnp.float32)
    n_slots = 6
    slots = _top8_per_lane(d, iota_f)[:n_slots]
    sv = [s[0] for s in slots]
    sg = [s[1] for s in slots]
    inf = jnp.float32(jnp.inf)
    for k in range(K):
        m = jnp.min(sv[0], axis=1, keepdims=True)
        cand = jnp.where(sv[0] == m, sg[0], BIG)
        idx_f = jnp.min(cand, axis=1, keepdims=True)
        idx_ref[0, :, k] = idx_f[:, 0].astype(jnp.int32)
        if k == K - 1:
            break
        w = cand == idx_f
        for s in range(n_slots - 1):
            sv[s] = jnp.where(w, sv[s + 1], sv[s])
            sg[s] = jnp.where(w, sg[s + 1], sg[s])
        sv[n_slots - 1] = jnp.where(w, inf, sv[n_slots - 1])
        sg[n_slots - 1] = jnp.where(w, BIG, sg[n_slots - 1])


def _knn_indices(h):
    n_samples = h.shape[0]
    x2 = jnp.sum(h * h, axis=-1)
    x2_r = x2[:, :, None]
    x2_c = x2[:, None, :]
    grid = (n_samples, N_POINTS // ROW_BLOCK)
    return pl.pallas_call(
        _knn_block_kernel,
        grid=grid,
        in_specs=[
            pl.BlockSpec((1, ROW_BLOCK, N_DIMS), lambda s, r: (s, r, 0)),
            pl.BlockSpec((1, N_POINTS, N_DIMS), lambda s, r: (s, 0, 0)),
            pl.BlockSpec((1, ROW_BLOCK, 1), lambda s, r: (s, r, 0)),
            pl.BlockSpec((1, 1, N_POINTS), lambda s, r: (s, 0, 0)),
        ],
        out_specs=pl.BlockSpec((1, ROW_BLOCK, K), lambda s, r: (s, r, 0)),
        out_shape=jax.ShapeDtypeStruct((n_samples, N_POINTS, K), jnp.int32),
    )(h, h, x2_r, x2_c)


def kernel(h):
    n_samples, n_points, _ = h.shape
    k_indices = _knn_indices(h)
    src = jnp.repeat(jnp.arange(n_points, dtype=jnp.int32), K)
    dst = k_indices.reshape(n_samples, n_points * K)
    return (k_indices, src, dst)


# speedup vs baseline: 18.2033x; 1.0855x over previous
import jax
import jax.numpy as jnp
from jax.experimental import pallas as pl

K = 16
N_POINTS = 2048
N_DIMS = 256
ROW_BLOCK = 512
N_CHUNKS = 16
LANES = N_POINTS // N_CHUNKS
BIG = 4096.0


def _ce(a, b):
    av, ag = a
    bv, bg = b
    swap = bv < av
    lo = (jnp.where(swap, bv, av), jnp.where(swap, bg, ag))
    hi = (jnp.where(swap, av, bv), jnp.where(swap, ag, bg))
    return lo, hi


def _cmin(a, b):
    av, ag = a
    bv, bg = b
    swap = bv < av
    return (jnp.where(swap, bv, av), jnp.where(swap, bg, ag))


def _oemerge(a, b):
    if len(a) == 1:
        lo, hi = _ce(a[0], b[0])
        return [lo, hi]
    ev = _oemerge(a[0::2], b[0::2])
    od = _oemerge(a[1::2], b[1::2])
    out = [ev[0]]
    for i in range(len(od) - 1):
        lo, hi = _ce(od[i], ev[i + 1])
        out.extend((lo, hi))
    out.append(od[-1])
    return out


def _bitonic_sort8(x):
    x = list(x)
    for stride, idxs in ((4, (0, 1, 2, 3)), (2, (0, 1, 4, 5)),
                         (1, (0, 2, 4, 6))):
        for i in idxs:
            x[i], x[i + stride] = _ce(x[i], x[i + stride])
    return x


def _top8_per_lane(d, iota_f):
    lists = [[(d[:, c * LANES:(c + 1) * LANES],
               iota_f[:, c * LANES:(c + 1) * LANES])]
             for c in range(N_CHUNKS)]
    while len(lists) > 2:
        lists = [_oemerge(lists[i], lists[i + 1])
                 for i in range(0, len(lists), 2)]
    a, b = lists
    lows = [_cmin(a[i], b[7 - i]) for i in range(8)]
    return _bitonic_sort8(lows)


def _knn_block_kernel(x_row_ref, x_all_ref, n_row_ref, n_all_ref, idx_ref):
    x_r = x_row_ref[0]
    x_c = x_all_ref[0]
    dot = jax.lax.dot_general(
        x_r, x_c, (((1,), (1,)), ((), ())),
        preferred_element_type=jnp.float32)
    n_r = n_row_ref[0]
    n_c = n_all_ref[0]
    d = (n_r + n_c) - 2.0 * dot

    iota_f = jax.lax.broadcasted_iota(
        jnp.int32, (ROW_BLOCK, N_POINTS), 1).astype(jnp.float32)
    n_slots = 6
    slots = _top8_per_lane(d, iota_f)[:n_slots]
    sv = [s[0] for s in slots]
    sg = [s[1] for s in slots]
    inf = jnp.float32(jnp.inf)
    for k in range(K):
        m = jnp.min(sv[0], axis=1, keepdims=True)
        cand = jnp.where(sv[0] == m, sg[0], BIG)
        idx_f = jnp.min(cand, axis=1, keepdims=True)
        idx_ref[0, :, k] = idx_f[:, 0].astype(jnp.int32)
        if k == K - 1:
            break
        w = cand == idx_f
        for s in range(n_slots - 1):
            sv[s] = jnp.where(w, sv[s + 1], sv[s])
            sg[s] = jnp.where(w, sg[s + 1], sg[s])
        sv[n_slots - 1] = jnp.where(w, inf, sv[n_slots - 1])
        sg[n_slots - 1] = jnp.where(w, BIG, sg[n_slots - 1])


def _knn_indices(h):
    n_samples = h.shape[0]
    x2 = jnp.sum(h * h, axis=-1)
    x2_r = x2[:, :, None]
    x2_c = x2[:, None, :]
    grid = (n_samples, N_POINTS // ROW_BLOCK)
    return pl.pallas_call(
        _knn_block_kernel,
        grid=grid,
        in_specs=[
            pl.BlockSpec((1, ROW_BLOCK, N_DIMS), lambda s, r: (s, r, 0)),
            pl.BlockSpec((1, N_POINTS, N_DIMS), lambda s, r: (s, 0, 0)),
            pl.BlockSpec((1, ROW_BLOCK, 1), lambda s, r: (s, r, 0)),
            pl.BlockSpec((1, 1, N_POINTS), lambda s, r: (s, 0, 0)),
        ],
        out_specs=pl.BlockSpec((1, ROW_BLOCK, K), lambda s, r: (s, r, 0)),
        out_shape=jax.ShapeDtypeStruct((n_samples, N_POINTS, K), jnp.int32),
    )(h, h, x2_r, x2_c)


def kernel(h):
    n_samples, n_points, _ = h.shape
    k_indices = _knn_indices(h)
    src = jnp.repeat(jnp.arange(n_points, dtype=jnp.int32), K)
    dst = k_indices.reshape(n_samples, n_points * K)
    return (k_indices, src, dst)
